# Initial kernel scaffold; baseline (speedup 1.0000x reference)
import jax
import jax.numpy as jnp
from jax.experimental import pallas as pl


def kernel(data, clusts, edge_index):
    raise NotImplementedError("write your pallas kernel here")


# speedup vs baseline: 2.7340x; 2.7340x over previous
import functools

import jax
import jax.numpy as jnp
from jax import lax
from jax.experimental import pallas as pl
from jax.experimental.pallas import tpu as pltpu
from jax.experimental.pallas import tpu_sc as plsc

_NC = 2
_NS = 16
_NW = _NC * _NS

_EB = 64
_ICH = 8
_BIG = 1 << 30


def _sc_gather_rows(table, idx, keep=None):
    n = idx.shape[0]
    d = table.shape[1]
    dk = d if keep is None else keep
    rpw = n // _NW
    mesh = plsc.VectorSubcoreMesh(core_axis_name="c", subcore_axis_name="s")

    @functools.partial(
        pl.kernel,
        mesh=mesh,
        out_type=jax.ShapeDtypeStruct((n, dk), table.dtype),
        scratch_types=[
            pltpu.VMEM((rpw,), jnp.int32),
            pltpu.VMEM((rpw, d), table.dtype),
            pltpu.SemaphoreType.DMA,
        ],
        compiler_params=pltpu.CompilerParams(use_tc_tiling_on_sc=False),
    )
    def gather_kernel(table_hbm, idx_hbm, out_hbm, idx_v, rows_v, sem):
        wid = lax.axis_index("s") * _NC + lax.axis_index("c")
        base = wid * rpw
        pltpu.sync_copy(idx_hbm.at[pl.ds(base, rpw)], idx_v)
        pltpu.async_copy(table_hbm.at[idx_v], rows_v, sem).wait()
        if dk == d:
            pltpu.sync_copy(rows_v, out_hbm.at[pl.ds(base, rpw)])
        else:
            pltpu.sync_copy(rows_v.at[:, pl.ds(0, dk)],
                            out_hbm.at[pl.ds(base, rpw)])

    return gather_kernel(table, idx)


def _feats_body(a0_ref, a1_ref, a2_ref, b0_ref, b1_ref, b2_ref, out_ref):
    a0 = a0_ref[...]
    a1 = a1_ref[...]
    a2 = a2_ref[...]
    b0 = b0_ref[...]
    b1 = b1_ref[...]
    b2 = b2_ref[...]
    eb, cp = a0.shape

    rmin = jnp.full((eb, 1), jnp.inf, dtype=jnp.float32)
    rpos = jnp.zeros((eb, 1), dtype=jnp.int32)
    for ic in range(cp // _ICH):
        lo = ic * _ICH
        s0 = a0[:, lo:lo + _ICH]
        s1 = a1[:, lo:lo + _ICH]
        s2 = a2[:, lo:lo + _ICH]
        d0 = s0[:, :, None] - b0[:, None, :]
        d1 = s1[:, :, None] - b1[:, None, :]
        d2c = s2[:, :, None] - b2[:, None, :]
        dsq = (d0 * d0 + d1 * d1) + d2c * d2c
        m1 = jnp.min(dsq, axis=2)
        cmin = jnp.min(m1, axis=1, keepdims=True)
        pf = (lo + lax.broadcasted_iota(jnp.int32, (_ICH, cp), 0)) * cp \
            + lax.broadcasted_iota(jnp.int32, (_ICH, cp), 1)
        masked = jnp.where(dsq == cmin[:, :, None], pf[None], _BIG)
        cpos = jnp.min(jnp.min(masked, axis=2), axis=1, keepdims=True)
        upd = cmin < rmin
        rmin = jnp.where(upd, cmin, rmin)
        rpos = jnp.where(upd, cpos, rpos)

    i1 = rpos // cp
    i2 = rpos - i1 * cp
    ks = lax.broadcasted_iota(jnp.int32, (eb, cp), 1)
    oh1 = (ks == i1).astype(jnp.float32)
    oh2 = (ks == i2).astype(jnp.float32)
    v10 = jnp.sum(oh1 * a0, axis=1, keepdims=True)
    v11 = jnp.sum(oh1 * a1, axis=1, keepdims=True)
    v12 = jnp.sum(oh1 * a2, axis=1, keepdims=True)
    v20 = jnp.sum(oh2 * b0, axis=1, keepdims=True)
    v21 = jnp.sum(oh2 * b1, axis=1, keepdims=True)
    v22 = jnp.sum(oh2 * b2, axis=1, keepdims=True)
    dx = v10 - v20
    dy = v11 - v21
    dz = v12 - v22
    lend = jnp.sqrt((dx * dx + dy * dy) + dz * dz)
    safe = jnp.maximum(lend, 1e-30)
    pos = lend > 0
    nx = jnp.where(pos, dx / safe, dx)
    ny = jnp.where(pos, dy / safe, dy)
    nz = jnp.where(pos, dz / safe, dz)
    out_ref[...] = jnp.concatenate(
        [v10, v11, v12, v20, v21, v22, nx, ny, nz, lend,
         nx * nx, nx * ny, nx * nz,
         ny * nx, ny * ny, ny * nz,
         nz * nx, nz * ny, nz * nz], axis=1)


def _edge_feats_tc(a0, a1, a2, b0, b1, b2):
    e, cp = a0.shape
    spec = pl.BlockSpec((_EB, cp), lambda i: (i, 0))
    return pl.pallas_call(
        _feats_body,
        grid=(e // _EB,),
        in_specs=[spec] * 6,
        out_specs=pl.BlockSpec((_EB, 19), lambda i: (i, 0)),
        out_shape=jax.ShapeDtypeStruct((e, 19), jnp.float32),
    )(a0, a1, a2, b0, b1, b2)


def kernel(data, clusts, edge_index):
    e = edge_index.shape[1]
    c = clusts.shape[1]
    voxels8 = jnp.pad(data.astype(jnp.float32), ((0, 0), (0, 3)))
    clusts_flat = clusts.reshape(-1).astype(jnp.int32)
    g = _sc_gather_rows(voxels8, clusts_flat, keep=4)
    g_rows = g.reshape(clusts.shape[0], c * 4)
    idx_e = jnp.concatenate(
        [edge_index[0].astype(jnp.int32), edge_index[1].astype(jnp.int32)])
    xe = _sc_gather_rows(g_rows, idx_e)
    xr = xe.reshape(2 * e, c, 4)
    x1 = xr[:e]
    x2 = xr[e:]
    return _edge_feats_tc(x1[:, :, 0], x1[:, :, 1], x1[:, :, 2],
                          x2[:, :, 0], x2[:, :, 1], x2[:, :, 2])
